# initial kernel scaffold (unmeasured)
import jax
import jax.numpy as jnp
from jax import lax
from jax.experimental import pallas as pl
from jax.experimental.pallas import tpu as pltpu

M = 4096
N = 4096
KSH = 2048
NYH = N // 2
NC = 8
CM = M // NC


def kernel(A, B):
    my_y = lax.axis_index("y")
    a16 = A.astype(jnp.bfloat16)
    b16 = lax.dynamic_slice(B, (0, my_y * NYH), (KSH, NYH)).astype(jnp.bfloat16)

    def body(a_ref, b_ref, out_ref, p_buf, rx_buf,
             sx_send, sx_recv, sy_send, sy_recv, s_copy):
        my_x = lax.axis_index("x")
        my_y = lax.axis_index("y")
        col0 = my_y * NYH
        x_nbr = (1 - my_x, my_y)
        y_nbr = (my_x, 1 - my_y)

        barrier = pltpu.get_barrier_semaphore()
        for nbr in (x_nbr, y_nbr):
            pl.semaphore_signal(barrier, inc=1, device_id=nbr,
                                device_id_type=pl.DeviceIdType.MESH)
        pl.semaphore_wait(barrier, 2)

        for i in range(NC):
            rows = pl.ds(i * CM, CM)
            slot = i % 2

            p = jnp.dot(a_ref[rows, :], b_ref[...],
                        preferred_element_type=jnp.float32)
            p_buf[slot] = p.astype(jnp.bfloat16)

            rdma_x = pltpu.make_async_remote_copy(
                src_ref=p_buf.at[slot],
                dst_ref=rx_buf.at[i],
                send_sem=sx_send.at[slot],
                recv_sem=sx_recv.at[i],
                device_id=x_nbr,
                device_id_type=pl.DeviceIdType.MESH,
            )
            rdma_x.start()
            rdma_x.wait()

            rx_buf[i] = p_buf[slot] + rx_buf[i]

            rdma_y = pltpu.make_async_remote_copy(
                src_ref=rx_buf.at[i],
                dst_ref=out_ref.at[rows, pl.ds(col0, NYH)],
                send_sem=sy_send.at[i],
                recv_sem=sy_recv.at[i],
                device_id=y_nbr,
                device_id_type=pl.DeviceIdType.MESH,
            )
            rdma_y.start()

            copy = pltpu.make_async_copy(
                rx_buf.at[i], out_ref.at[rows, pl.ds(col0, NYH)], s_copy.at[i]
            )
            copy.start()
            rdma_y.wait()
            copy.wait()

    return pl.pallas_call(
        body,
        out_shape=jax.ShapeDtypeStruct((M, N), jnp.bfloat16),
        in_specs=[
            pl.BlockSpec(memory_space=pltpu.VMEM),
            pl.BlockSpec(memory_space=pltpu.VMEM),
        ],
        out_specs=pl.BlockSpec(memory_space=pl.ANY),
        scratch_shapes=[
            pltpu.VMEM((2, CM, NYH), jnp.bfloat16),
            pltpu.VMEM((NC, CM, NYH), jnp.bfloat16),
            pltpu.SemaphoreType.DMA((2,)),
            pltpu.SemaphoreType.DMA((NC,)),
            pltpu.SemaphoreType.DMA((NC,)),
            pltpu.SemaphoreType.DMA((NC,)),
            pltpu.SemaphoreType.DMA((NC,)),
        ],
        compiler_params=pltpu.CompilerParams(collective_id=0),
    )(a16, b16)


# baseline (device time: 490131 ns/iter reference)
import jax
import jax.numpy as jnp
from jax import lax
from jax.experimental import pallas as pl
from jax.experimental.pallas import tpu as pltpu

M = 4096
N = 4096
KSH = 2048
NYH = N // 2
NC = 8
CM = M // NC


def kernel(A, B):
    my_y = lax.axis_index("y")
    a16 = A.astype(jnp.bfloat16)
    b16 = lax.dynamic_slice(B, (0, my_y * NYH), (KSH, NYH)).astype(jnp.bfloat16)

    def body(a_ref, b_ref, out_ref, p_buf, rx_buf,
             sx_send, sx_recv, sy_send, sy_recv, s_copy):
        my_x = lax.axis_index("x")
        my_y = lax.axis_index("y")
        col0 = my_y * NYH
        x_nbr = (1 - my_x, my_y)
        y_nbr = (my_x, 1 - my_y)

        barrier = pltpu.get_barrier_semaphore()
        for nbr in (x_nbr, y_nbr):
            pl.semaphore_signal(barrier, inc=1, device_id=nbr,
                                device_id_type=pl.DeviceIdType.MESH)
        pl.semaphore_wait(barrier, 2)

        for i in range(NC):
            rows = pl.ds(i * CM, CM)
            slot = i % 2

            p = jnp.dot(a_ref[rows, :], b_ref[...],
                        preferred_element_type=jnp.float32)
            p_buf[slot] = p.astype(jnp.bfloat16)

            rdma_x = pltpu.make_async_remote_copy(
                src_ref=p_buf.at[slot],
                dst_ref=rx_buf.at[i],
                send_sem=sx_send.at[slot],
                recv_sem=sx_recv.at[i],
                device_id=x_nbr,
                device_id_type=pl.DeviceIdType.MESH,
            )
            rdma_x.start()
            rdma_x.wait()

            rx_buf[i] = p_buf[slot] + rx_buf[i]

            rdma_y = pltpu.make_async_remote_copy(
                src_ref=rx_buf.at[i],
                dst_ref=out_ref.at[rows, pl.ds(col0, NYH)],
                send_sem=sy_send.at[i],
                recv_sem=sy_recv.at[i],
                device_id=y_nbr,
                device_id_type=pl.DeviceIdType.MESH,
            )
            rdma_y.start()

            copy = pltpu.make_async_copy(
                rx_buf.at[i], out_ref.at[rows, pl.ds(col0, NYH)], s_copy.at[i]
            )
            copy.start()
            rdma_y.wait()
            copy.wait()

    return pl.pallas_call(
        body,
        out_shape=jax.ShapeDtypeStruct((M, N), jnp.bfloat16),
        in_specs=[
            pl.BlockSpec(memory_space=pltpu.VMEM),
            pl.BlockSpec(memory_space=pltpu.VMEM),
        ],
        out_specs=pl.BlockSpec(memory_space=pl.ANY),
        scratch_shapes=[
            pltpu.VMEM((2, CM, NYH), jnp.bfloat16),
            pltpu.VMEM((NC, CM, NYH), jnp.bfloat16),
            pltpu.SemaphoreType.DMA((2,)),
            pltpu.SemaphoreType.DMA((NC,)),
            pltpu.SemaphoreType.DMA((NC,)),
            pltpu.SemaphoreType.DMA((NC,)),
            pltpu.SemaphoreType.DMA((NC,)),
        ],
        compiler_params=pltpu.CompilerParams(
            collective_id=0, vmem_limit_bytes=60 * 1024 * 1024
        ),
    )(a16, b16)


# device time: 280765 ns/iter; 1.7457x vs baseline; 1.7457x over previous
import jax
import jax.numpy as jnp
from jax import lax
from jax.experimental import pallas as pl
from jax.experimental.pallas import tpu as pltpu

M = 4096
N = 4096
KSH = 2048
NYH = N // 2
NC = 8
CM = M // NC


def kernel(A, B):
    my_y = lax.axis_index("y")
    a16 = A.astype(jnp.bfloat16)
    b16 = lax.dynamic_slice(B, (0, my_y * NYH), (KSH, NYH)).astype(jnp.bfloat16)

    def body(a_ref, b_ref, out_ref, p_buf, rx_buf,
             sx_send, sx_recv, sy_send, sy_recv, s_copy):
        my_x = lax.axis_index("x")
        my_y = lax.axis_index("y")
        col0 = my_y * NYH
        x_nbr = (1 - my_x, my_y)
        y_nbr = (my_x, 1 - my_y)

        barrier = pltpu.get_barrier_semaphore()
        for nbr in (x_nbr, y_nbr):
            pl.semaphore_signal(barrier, inc=1, device_id=nbr,
                                device_id_type=pl.DeviceIdType.MESH)
        pl.semaphore_wait(barrier, 2)

        x_descs = []
        for i in range(NC):
            rows = pl.ds(i * CM, CM)
            p = jnp.dot(a_ref[rows, :], b_ref[...],
                        preferred_element_type=jnp.float32)
            p_buf[i] = p.astype(jnp.bfloat16)
            rdma_x = pltpu.make_async_remote_copy(
                src_ref=p_buf.at[i],
                dst_ref=rx_buf.at[i],
                send_sem=sx_send.at[i],
                recv_sem=sx_recv.at[i],
                device_id=x_nbr,
                device_id_type=pl.DeviceIdType.MESH,
            )
            rdma_x.start()
            x_descs.append(rdma_x)

        y_descs = []
        copies = []
        for i in range(NC):
            rows = pl.ds(i * CM, CM)
            x_descs[i].wait_recv()
            rx_buf[i] = p_buf[i] + rx_buf[i]
            rdma_y = pltpu.make_async_remote_copy(
                src_ref=rx_buf.at[i],
                dst_ref=out_ref.at[rows, pl.ds(col0, NYH)],
                send_sem=sy_send.at[i],
                recv_sem=sy_recv.at[i],
                device_id=y_nbr,
                device_id_type=pl.DeviceIdType.MESH,
            )
            rdma_y.start()
            y_descs.append(rdma_y)
            copy = pltpu.make_async_copy(
                rx_buf.at[i], out_ref.at[rows, pl.ds(col0, NYH)], s_copy.at[i]
            )
            copy.start()
            copies.append(copy)

        for i in range(NC):
            x_descs[i].wait_send()
            y_descs[i].wait_send()
            y_descs[i].wait_recv()
            copies[i].wait()

    return pl.pallas_call(
        body,
        out_shape=jax.ShapeDtypeStruct((M, N), jnp.bfloat16),
        in_specs=[
            pl.BlockSpec(memory_space=pltpu.VMEM),
            pl.BlockSpec(memory_space=pltpu.VMEM),
        ],
        out_specs=pl.BlockSpec(memory_space=pl.ANY),
        scratch_shapes=[
            pltpu.VMEM((NC, CM, NYH), jnp.bfloat16),
            pltpu.VMEM((NC, CM, NYH), jnp.bfloat16),
            pltpu.SemaphoreType.DMA((NC,)),
            pltpu.SemaphoreType.DMA((NC,)),
            pltpu.SemaphoreType.DMA((NC,)),
            pltpu.SemaphoreType.DMA((NC,)),
            pltpu.SemaphoreType.DMA((NC,)),
        ],
        compiler_params=pltpu.CompilerParams(
            collective_id=0, vmem_limit_bytes=60 * 1024 * 1024
        ),
    )(a16, b16)


# device time: 263183 ns/iter; 1.8623x vs baseline; 1.0668x over previous
import jax
import jax.numpy as jnp
from jax import lax
from jax.experimental import pallas as pl
from jax.experimental.pallas import tpu as pltpu

M = 4096
N = 4096
KSH = 2048
NYH = N // 2
NC = 16
CM = M // NC


def kernel(A, B):
    my_y = lax.axis_index("y")
    b16 = lax.dynamic_slice(B, (0, my_y * NYH), (KSH, NYH)).astype(jnp.bfloat16)

    def body(a_hbm, b_ref, out_ref, a_slots, p_buf, rx_buf,
             sa, sx_send, sx_recv, sy_send, sy_recv, s_copy):
        my_x = lax.axis_index("x")
        my_y = lax.axis_index("y")
        col0 = my_y * NYH
        x_nbr = (1 - my_x, my_y)
        y_nbr = (my_x, 1 - my_y)

        def a_copy(i, slot):
            return pltpu.make_async_copy(
                a_hbm.at[pl.ds(i * CM, CM), :], a_slots.at[slot], sa.at[slot]
            )

        barrier = pltpu.get_barrier_semaphore()
        for nbr in (x_nbr, y_nbr):
            pl.semaphore_signal(barrier, inc=1, device_id=nbr,
                                device_id_type=pl.DeviceIdType.MESH)
        pl.semaphore_wait(barrier, 2)

        a_copy(0, 0).start()
        x_descs = []
        for i in range(NC):
            slot = i % 2
            if i + 1 < NC:
                a_copy(i + 1, 1 - slot).start()
            a_copy(i, slot).wait()
            p = jnp.dot(a_slots[slot].astype(jnp.bfloat16), b_ref[...],
                        preferred_element_type=jnp.float32)
            p_buf[i] = p.astype(jnp.bfloat16)
            rdma_x = pltpu.make_async_remote_copy(
                src_ref=p_buf.at[i],
                dst_ref=rx_buf.at[i],
                send_sem=sx_send.at[i],
                recv_sem=sx_recv.at[i],
                device_id=x_nbr,
                device_id_type=pl.DeviceIdType.MESH,
            )
            rdma_x.start()
            x_descs.append(rdma_x)

        y_descs = []
        copies = []
        for i in range(NC):
            rows = pl.ds(i * CM, CM)
            x_descs[i].wait_recv()
            rx_buf[i] = p_buf[i] + rx_buf[i]
            rdma_y = pltpu.make_async_remote_copy(
                src_ref=rx_buf.at[i],
                dst_ref=out_ref.at[rows, pl.ds(col0, NYH)],
                send_sem=sy_send.at[i],
                recv_sem=sy_recv.at[i],
                device_id=y_nbr,
                device_id_type=pl.DeviceIdType.MESH,
            )
            rdma_y.start()
            y_descs.append(rdma_y)
            copy = pltpu.make_async_copy(
                rx_buf.at[i], out_ref.at[rows, pl.ds(col0, NYH)], s_copy.at[i]
            )
            copy.start()
            copies.append(copy)

        for i in range(NC):
            x_descs[i].wait_send()
            y_descs[i].wait_send()
            y_descs[i].wait_recv()
            copies[i].wait()

    return pl.pallas_call(
        body,
        out_shape=jax.ShapeDtypeStruct((M, N), jnp.bfloat16),
        in_specs=[
            pl.BlockSpec(memory_space=pl.ANY),
            pl.BlockSpec(memory_space=pltpu.VMEM),
        ],
        out_specs=pl.BlockSpec(memory_space=pl.ANY),
        scratch_shapes=[
            pltpu.VMEM((2, CM, KSH), jnp.float32),
            pltpu.VMEM((NC, CM, NYH), jnp.bfloat16),
            pltpu.VMEM((NC, CM, NYH), jnp.bfloat16),
            pltpu.SemaphoreType.DMA((2,)),
            pltpu.SemaphoreType.DMA((NC,)),
            pltpu.SemaphoreType.DMA((NC,)),
            pltpu.SemaphoreType.DMA((NC,)),
            pltpu.SemaphoreType.DMA((NC,)),
            pltpu.SemaphoreType.DMA((NC,)),
        ],
        compiler_params=pltpu.CompilerParams(
            collective_id=0, vmem_limit_bytes=60 * 1024 * 1024
        ),
    )(A, b16)
